# baseline (speedup 1.0000x reference)
import functools

import jax
import jax.numpy as jnp
from jax import lax
from jax.experimental import pallas as pl
from jax.experimental.pallas import tpu as pltpu
from jax.experimental.pallas import tpu_sc as plsc

B = 16384
D = 64
WL = 16
NC = 2
NS = 16
NW = NC * NS
BPW = B // NW
CH = 128
NCH = BPW // CH
L = 16
NBUF = 4
DDH = BPW // 2


def _sc_deep_body(uids, iids, deep_u, deep_i, wide_u, wide_i,
                  dd_out, w_out,
                  uidx_s, iidx_s, uidx_vm, iidx_vm, slot_u, slot_i,
                  wslot_u, wslot_i, dd_v, wout_vm,
                  sem0, sem1, sem2, sem3):
    wid = lax.axis_index("s") * NC + lax.axis_index("c")
    base = wid * BPW
    pltpu.sync_copy(uids.at[pl.ds(base, BPW)], uidx_vm)
    pltpu.sync_copy(iids.at[pl.ds(base, BPW)], iidx_vm)

    def fill(k, carry):
        u16 = uidx_vm[pl.ds(k * L, L)]
        i16 = iidx_vm[pl.ds(k * L, L)]
        for l in range(L):
            uidx_s[k * L + l] = u16[l]
            iidx_s[k * L + l] = i16[l]
        return carry

    lax.fori_loop(0, BPW // L, fill, 0)
    sems = (sem0, sem1, sem2, sem3)

    def fire(b, s):
        ublk = lax.shift_right_logical(uidx_s[b], 7)
        iblk = lax.shift_right_logical(iidx_s[b], 7)
        uoff = pl.multiple_of(ublk * CH, CH)
        ioff = pl.multiple_of(iblk * CH, CH)
        pltpu.async_copy(deep_u.at[:, pl.ds(uoff, CH)], slot_u.at[s], sems[s])
        pltpu.async_copy(deep_i.at[:, pl.ds(ioff, CH)], slot_i.at[s], sems[s])
        pltpu.async_copy(wide_u.at[:, pl.ds(uoff, CH)], wslot_u.at[s], sems[s])
        pltpu.async_copy(wide_i.at[:, pl.ds(ioff, CH)], wslot_i.at[s], sems[s])

    def consume(b, s):
        pltpu.make_async_copy(deep_u.at[:, pl.ds(0, CH)], slot_u.at[s], sems[s]).wait()
        pltpu.make_async_copy(deep_i.at[:, pl.ds(0, CH)], slot_i.at[s], sems[s]).wait()
        pltpu.make_async_copy(wide_u.at[:, pl.ds(0, CH)], wslot_u.at[s], sems[s]).wait()
        pltpu.make_async_copy(wide_i.at[:, pl.ds(0, CH)], wslot_i.at[s], sems[s]).wait()
        uc = jnp.bitwise_and(uidx_s[b], CH - 1)
        ic = jnp.bitwise_and(iidx_s[b], CH - 1)
        ucols = jnp.full((L,), uc, jnp.int32)
        icols = jnp.full((L,), ic, jnp.int32)
        bh = jnp.bitwise_and(b, DDH - 1)
        for k in range(D // L):
            rows = lax.iota(jnp.int32, L) + (k * L)
            dd_v[bh, pl.ds(k * L, L)] = plsc.load_gather(slot_u.at[s], [rows, ucols])
            dd_v[bh, pl.ds(D + k * L, L)] = plsc.load_gather(slot_i.at[s], [rows, icols])
        zrows = jnp.zeros((L,), jnp.int32)
        wval = (plsc.load_gather(wslot_u.at[s], [zrows, ucols])
                + plsc.load_gather(wslot_i.at[s], [zrows, icols]))
        plsc.store_scatter(
            wout_vm,
            [jnp.full((L,), lax.shift_right_logical(b, 7), jnp.int32),
             jnp.full((L,), jnp.bitwise_and(b, CH - 1), jnp.int32)],
            wval, mask=lax.iota(jnp.int32, L) == 0)

    for s in range(NBUF):
        fire(s, s)

    def body(g, carry):
        b = g * NBUF
        for s in range(NBUF):
            consume(b + s, s)
            fire(b + s + NBUF, s)

        @pl.when(b + NBUF == DDH)
        def _flush_first():
            pltpu.sync_copy(dd_v, dd_out.at[wid, pl.ds(0, DDH)])

        return carry

    lax.fori_loop(0, BPW // NBUF - 1, body, 0)
    for s in range(NBUF):
        consume(BPW - NBUF + s, s)
    pltpu.sync_copy(dd_v, dd_out.at[wid, pl.ds(DDH, DDH)])
    for j in range(BPW // CH):
        pltpu.sync_copy(wout_vm.at[j], w_out.at[wid, 0, pl.ds(j * CH, CH)])


@functools.lru_cache(maxsize=1)
def _build_sc_deep():
    return functools.partial(
        pl.kernel,
        out_type=(
            jax.ShapeDtypeStruct((NW, BPW, 2 * D), jnp.float32),
            jax.ShapeDtypeStruct((NW, 1, BPW), jnp.float32),
        ),
        mesh=plsc.VectorSubcoreMesh(
            core_axis_name="c", subcore_axis_name="s", num_cores=NC, num_subcores=NS
        ),
        scratch_types=(
            pltpu.SMEM((BPW,), jnp.int32),
            pltpu.SMEM((BPW,), jnp.int32),
            pltpu.VMEM((BPW,), jnp.int32),
            pltpu.VMEM((BPW,), jnp.int32),
            pltpu.VMEM((NBUF, D, CH), jnp.float32),
            pltpu.VMEM((NBUF, D, CH), jnp.float32),
            pltpu.VMEM((NBUF, 1, CH), jnp.float32),
            pltpu.VMEM((NBUF, 1, CH), jnp.float32),
            pltpu.VMEM((DDH, 2 * D), jnp.float32),
            pltpu.VMEM((BPW // CH, CH), jnp.float32),
            pltpu.SemaphoreType.DMA,
            pltpu.SemaphoreType.DMA,
            pltpu.SemaphoreType.DMA,
            pltpu.SemaphoreType.DMA,
        ),
        compiler_params=pltpu.CompilerParams(
            use_tc_tiling_on_sc=True, needs_layout_passes=False),
    )(_sc_deep_body)


def _mlp_body(dd_ref, w_ref, w0_ref, b0_ref, w1_ref, b1_ref,
              w2_ref, b2_ref, w3_ref, b3_ref, out_ref):
    hp = lax.Precision.HIGHEST
    x = dd_ref[0]
    x = jax.nn.relu(jnp.dot(x, w0_ref[...], preferred_element_type=jnp.float32,
                            precision=hp) + b0_ref[...])
    x = jax.nn.relu(jnp.dot(x, w1_ref[...], preferred_element_type=jnp.float32,
                            precision=hp) + b1_ref[...])
    x = jax.nn.relu(jnp.dot(x, w2_ref[...], preferred_element_type=jnp.float32,
                            precision=hp) + b2_ref[...])
    deep = jnp.dot(x, w3_ref[...], preferred_element_type=jnp.float32, precision=hp)
    out_ref[0, 0] = deep[:, 0] + b3_ref[0, 0] + w_ref[0, 0]


def _mlp_call(dd, w, w0, b0, w1, b1, w2, b2, w3, b3):
    full = lambda shape: pl.BlockSpec(shape, lambda i: (0,) * len(shape))
    return pl.pallas_call(
        _mlp_body,
        grid=(NW,),
        in_specs=[
            pl.BlockSpec((1, BPW, 2 * D), lambda i: (i, 0, 0)),
            pl.BlockSpec((1, 1, BPW), lambda i: (i, 0, 0)),
            full((128, 128)),
            full((1, 128)),
            full((128, 64)),
            full((1, 64)),
            full((64, 32)),
            full((1, 32)),
            full((32, 1)),
            full((1, 1)),
        ],
        out_specs=pl.BlockSpec((1, 1, BPW), lambda i: (i, 0, 0)),
        out_shape=jax.ShapeDtypeStruct((NW, 1, BPW), jnp.float32),
    )(dd, w, w0, b0, w1, b1, w2, b2, w3, b3)


def kernel(user_ids, item_ids, wide_user, wide_item, deep_user, deep_item,
           W0, b0, W1, b1, W2, b2, W3, b3):
    uids = user_ids.astype(jnp.int32)
    iids = item_ids.astype(jnp.int32)
    dd, w = _build_sc_deep()(uids, iids, deep_user.T, deep_item.T,
                             wide_user.T, wide_item.T)
    out2 = _mlp_call(
        dd, w,
        W0.T, b0.reshape(1, -1),
        W1.T, b1.reshape(1, -1),
        W2.T, b2.reshape(1, -1),
        W3.T, b3.reshape(1, 1),
    )
    return out2.reshape(B)


# speedup vs baseline: 1.0957x; 1.0957x over previous
import functools

import jax
import jax.numpy as jnp
from jax import lax
from jax.experimental import pallas as pl
from jax.experimental.pallas import tpu as pltpu
from jax.experimental.pallas import tpu_sc as plsc

B = 16384
D = 64
WL = 16
NC = 2
NS = 16
NW = NC * NS
BPW = B // NW
CH = 128
NCH = BPW // CH
L = 16
NBUF = 4
DDH = BPW // 2


def _sc_deep_body(uids, iids, deep_u, deep_i, wide_u, wide_i,
                  dd_out, w_out,
                  uidx_s, iidx_s, uidx_vm, iidx_vm, slot_u, slot_i,
                  wslot_u, wslot_i, dd_v, wout_vm,
                  sem0, sem1, sem2, sem3):
    wid = lax.axis_index("s") * NC + lax.axis_index("c")
    base = wid * BPW
    pltpu.sync_copy(uids.at[pl.ds(base, BPW)], uidx_vm)
    pltpu.sync_copy(iids.at[pl.ds(base, BPW)], iidx_vm)

    def fill(k, carry):
        u16 = uidx_vm[pl.ds(k * L, L)]
        i16 = iidx_vm[pl.ds(k * L, L)]
        for l in range(L):
            uidx_s[k * L + l] = u16[l]
            iidx_s[k * L + l] = i16[l]
        return carry

    lax.fori_loop(0, BPW // L, fill, 0)
    sems = (sem0, sem1, sem2, sem3)

    def fire(b, s):
        ublk = lax.shift_right_logical(uidx_s[b], 7)
        iblk = lax.shift_right_logical(iidx_s[b], 7)
        uoff = pl.multiple_of(ublk * CH, CH)
        ioff = pl.multiple_of(iblk * CH, CH)
        pltpu.async_copy(deep_u.at[:, pl.ds(uoff, CH)], slot_u.at[s], sems[s])
        pltpu.async_copy(deep_i.at[:, pl.ds(ioff, CH)], slot_i.at[s], sems[s])
        pltpu.async_copy(wide_u.at[:, pl.ds(uoff, CH)], wslot_u.at[s], sems[s])
        pltpu.async_copy(wide_i.at[:, pl.ds(ioff, CH)], wslot_i.at[s], sems[s])

    def consume(b, s):
        pltpu.make_async_copy(deep_u.at[:, pl.ds(0, CH)], slot_u.at[s], sems[s]).wait()
        pltpu.make_async_copy(deep_i.at[:, pl.ds(0, CH)], slot_i.at[s], sems[s]).wait()
        pltpu.make_async_copy(wide_u.at[:, pl.ds(0, CH)], wslot_u.at[s], sems[s]).wait()
        pltpu.make_async_copy(wide_i.at[:, pl.ds(0, CH)], wslot_i.at[s], sems[s]).wait()
        uc = jnp.bitwise_and(uidx_s[b], CH - 1)
        ic = jnp.bitwise_and(iidx_s[b], CH - 1)
        ucols = jnp.full((L,), uc, jnp.int32)
        icols = jnp.full((L,), ic, jnp.int32)
        bh = jnp.bitwise_and(b, DDH - 1)
        for k in range(D // L):
            rows = lax.iota(jnp.int32, L) + (k * L)
            dd_v[bh, pl.ds(k * L, L)] = plsc.load_gather(slot_u.at[s], [rows, ucols])
            dd_v[bh, pl.ds(D + k * L, L)] = plsc.load_gather(slot_i.at[s], [rows, icols])
        zrows = jnp.zeros((L,), jnp.int32)
        wval = (plsc.load_gather(wslot_u.at[s], [zrows, ucols])
                + plsc.load_gather(wslot_i.at[s], [zrows, icols]))
        plsc.store_scatter(
            wout_vm,
            [jnp.full((L,), lax.shift_right_logical(b, 7), jnp.int32),
             jnp.full((L,), jnp.bitwise_and(b, CH - 1), jnp.int32)],
            wval, mask=lax.iota(jnp.int32, L) == 0)

    for s in range(NBUF):
        fire(s, s)

    def body(g, carry):
        b = g * NBUF
        for s in range(NBUF):
            consume(b + s, s)
            fire(b + s + NBUF, s)

        @pl.when(b + NBUF == DDH)
        def _flush_first():
            pltpu.sync_copy(dd_v, dd_out.at[wid, pl.ds(0, DDH)])

        return carry

    lax.fori_loop(0, BPW // NBUF - 1, body, 0)
    for s in range(NBUF):
        consume(BPW - NBUF + s, s)
    pltpu.sync_copy(dd_v, dd_out.at[wid, pl.ds(DDH, DDH)])
    for j in range(BPW // CH):
        pltpu.sync_copy(wout_vm.at[j], w_out.at[wid, 0, pl.ds(j * CH, CH)])


@functools.lru_cache(maxsize=1)
def _build_sc_deep():
    return functools.partial(
        pl.kernel,
        out_type=(
            jax.ShapeDtypeStruct((NW, BPW, 2 * D), jnp.float32),
            jax.ShapeDtypeStruct((NW, 1, BPW), jnp.float32),
        ),
        mesh=plsc.VectorSubcoreMesh(
            core_axis_name="c", subcore_axis_name="s", num_cores=NC, num_subcores=NS
        ),
        scratch_types=(
            pltpu.SMEM((BPW,), jnp.int32),
            pltpu.SMEM((BPW,), jnp.int32),
            pltpu.VMEM((BPW,), jnp.int32),
            pltpu.VMEM((BPW,), jnp.int32),
            pltpu.VMEM((NBUF, D, CH), jnp.float32),
            pltpu.VMEM((NBUF, D, CH), jnp.float32),
            pltpu.VMEM((NBUF, 1, CH), jnp.float32),
            pltpu.VMEM((NBUF, 1, CH), jnp.float32),
            pltpu.VMEM((DDH, 2 * D), jnp.float32),
            pltpu.VMEM((BPW // CH, CH), jnp.float32),
            pltpu.SemaphoreType.DMA,
            pltpu.SemaphoreType.DMA,
            pltpu.SemaphoreType.DMA,
            pltpu.SemaphoreType.DMA,
        ),
        compiler_params=pltpu.CompilerParams(
            use_tc_tiling_on_sc=True, needs_layout_passes=False),
    )(_sc_deep_body)


def _mlp_body(dd_ref, w_ref, w0_ref, b0_ref, w1_ref, b1_ref,
              w2_ref, b2_ref, w3_ref, b3_ref, out_ref):
    hp = lax.Precision.DEFAULT
    x = dd_ref[0]
    x = jax.nn.relu(jnp.dot(x, w0_ref[...], preferred_element_type=jnp.float32,
                            precision=hp) + b0_ref[...])
    x = jax.nn.relu(jnp.dot(x, w1_ref[...], preferred_element_type=jnp.float32,
                            precision=hp) + b1_ref[...])
    x = jax.nn.relu(jnp.dot(x, w2_ref[...], preferred_element_type=jnp.float32,
                            precision=hp) + b2_ref[...])
    deep = jnp.dot(x, w3_ref[...], preferred_element_type=jnp.float32, precision=hp)
    out_ref[0, 0] = deep[:, 0] + b3_ref[0, 0] + w_ref[0, 0]


def _mlp_call(dd, w, w0, b0, w1, b1, w2, b2, w3, b3):
    full = lambda shape: pl.BlockSpec(shape, lambda i: (0,) * len(shape))
    return pl.pallas_call(
        _mlp_body,
        grid=(NW,),
        in_specs=[
            pl.BlockSpec((1, BPW, 2 * D), lambda i: (i, 0, 0)),
            pl.BlockSpec((1, 1, BPW), lambda i: (i, 0, 0)),
            full((128, 128)),
            full((1, 128)),
            full((128, 64)),
            full((1, 64)),
            full((64, 32)),
            full((1, 32)),
            full((32, 1)),
            full((1, 1)),
        ],
        out_specs=pl.BlockSpec((1, 1, BPW), lambda i: (i, 0, 0)),
        out_shape=jax.ShapeDtypeStruct((NW, 1, BPW), jnp.float32),
    )(dd, w, w0, b0, w1, b1, w2, b2, w3, b3)


def kernel(user_ids, item_ids, wide_user, wide_item, deep_user, deep_item,
           W0, b0, W1, b1, W2, b2, W3, b3):
    uids = user_ids.astype(jnp.int32)
    iids = item_ids.astype(jnp.int32)
    dd, w = _build_sc_deep()(uids, iids, deep_user.T, deep_item.T,
                             wide_user.T, wide_item.T)
    out2 = _mlp_call(
        dd, w,
        W0.T, b0.reshape(1, -1),
        W1.T, b1.reshape(1, -1),
        W2.T, b2.reshape(1, -1),
        W3.T, b3.reshape(1, 1),
    )
    return out2.reshape(B)


# speedup vs baseline: 1.8562x; 1.6941x over previous
import functools

import jax
import jax.numpy as jnp
from jax import lax
from jax.experimental import pallas as pl
from jax.experimental.pallas import tpu as pltpu
from jax.experimental.pallas import tpu_sc as plsc

B = 16384
D = 64
NC = 2
NS = 16
NW = NC * NS
BPW = B // NW
CH = 128
L = 16
NBUF = 4
BLK = 512


def _pass_body(sids, perm, deep_t, wide_t, row_out,
               sid_vm, perm_vm, blk_s, sid_s, slot, wslot, row_v,
               sem_arr, base):
    pltpu.sync_copy(sids.at[pl.ds(base, BPW)], sid_vm)
    for j in range(BPW // CH):
        pltpu.sync_copy(perm.at[pl.ds(base + j * CH, CH)], perm_vm.at[j])

    def fill(k, carry):
        v16 = sid_vm[pl.ds(k * L, L)]
        for l in range(L):
            sid_s[k * L + l] = v16[l]
        return carry

    lax.fori_loop(0, BPW // L, fill, 0)

    def scan(b, carry):
        prev, nu = carry
        blk = lax.shift_right_logical(sid_s[b], 7)
        is_new = jnp.logical_or(b == 0, blk != prev)
        nu_new = jnp.where(is_new, nu + 1, nu)

        @pl.when(is_new)
        def _record():
            blk_s[nu] = blk

        return blk, nu_new

    _, nu = lax.fori_loop(0, BPW, scan, (jnp.int32(-1), jnp.int32(0)))

    def fire(k):
        off = pl.multiple_of(blk_s[k] * CH, CH)
        s = jnp.bitwise_and(k, NBUF - 1)
        pltpu.async_copy(deep_t.at[:, pl.ds(off, CH)], slot.at[s], sem_arr.at[s])
        pltpu.async_copy(wide_t.at[:, pl.ds(off, CH)], wslot.at[s], sem_arr.at[s])

    for j in range(NBUF):
        @pl.when(j < nu)
        def _prime():
            fire(jnp.int32(j))

    def body(b, carry):
        k, prev = carry
        blk = lax.shift_right_logical(sid_s[b], 7)
        is_new = jnp.logical_or(b == 0, blk != prev)
        k = jnp.where(is_new, k + 1, k)
        s = jnp.bitwise_and(k, NBUF - 1)

        @pl.when(is_new)
        def _advance():
            @pl.when(jnp.logical_and(k > 0, k + NBUF - 1 < nu))
            def _refill():
                fire(k + NBUF - 1)

            pltpu.make_async_copy(deep_t.at[:, pl.ds(0, CH)], slot.at[s], sem_arr.at[s]).wait()
            pltpu.make_async_copy(wide_t.at[:, pl.ds(0, CH)], wslot.at[s], sem_arr.at[s]).wait()

        c = jnp.bitwise_and(sid_s[b], CH - 1)
        cols = jnp.full((L,), c, jnp.int32)
        for kk in range(D // L):
            rows = lax.iota(jnp.int32, L) + (kk * L)
            row_v[b, pl.ds(kk * L, L)] = plsc.load_gather(slot.at[s], [rows, cols])
        wv = plsc.load_gather(wslot.at[s], [jnp.zeros((L,), jnp.int32), cols])
        row_v[b, pl.ds(D, L)] = wv
        return k, blk

    lax.fori_loop(0, BPW, body, (jnp.int32(-1), jnp.int32(-1)))

    copies = []
    for j in range(BPW // CH):
        copies.append(pltpu.async_copy(
            row_v.at[pl.ds(j * CH, CH)], row_out.at[perm_vm.at[j]], sem_arr.at[j]))
    for c in copies:
        c.wait()


def _sc_deep_body(su, pu, si, pi, deep_u, deep_i, wide_u, wide_i,
                  du_out, di_out,
                  sid_vm, perm_vm, blk_s, sid_s, slot, wslot, row_v, sem_arr):
    wid = lax.axis_index("s") * NC + lax.axis_index("c")
    base = wid * BPW
    _pass_body(su, pu, deep_u, wide_u, du_out,
               sid_vm, perm_vm, blk_s, sid_s, slot, wslot, row_v, sem_arr, base)
    _pass_body(si, pi, deep_i, wide_i, di_out,
               sid_vm, perm_vm, blk_s, sid_s, slot, wslot, row_v, sem_arr, base)


@functools.lru_cache(maxsize=1)
def _build_sc_deep():
    return functools.partial(
        pl.kernel,
        out_type=(
            jax.ShapeDtypeStruct((B, 2 * D), jnp.float32),
            jax.ShapeDtypeStruct((B, 2 * D), jnp.float32),
        ),
        mesh=plsc.VectorSubcoreMesh(
            core_axis_name="c", subcore_axis_name="s", num_cores=NC, num_subcores=NS
        ),
        scratch_types=(
            pltpu.VMEM((BPW,), jnp.int32),
            pltpu.VMEM((BPW // CH, CH), jnp.int32),
            pltpu.SMEM((BPW,), jnp.int32),
            pltpu.SMEM((BPW,), jnp.int32),
            pltpu.VMEM((NBUF, D, CH), jnp.float32),
            pltpu.VMEM((NBUF, 1, CH), jnp.float32),
            pltpu.VMEM((BPW, 2 * D), jnp.float32),
            pltpu.SemaphoreType.DMA((NBUF,)),
        ),
        compiler_params=pltpu.CompilerParams(
            use_tc_tiling_on_sc=True, needs_layout_passes=False),
    )(_sc_deep_body)


def _mlp_body(du_ref, di_ref, w0_ref, b0_ref, w1_ref, b1_ref,
              w2_ref, b2_ref, w3_ref, b3_ref, out_ref):
    du = du_ref[...]
    di = di_ref[...]
    x = jax.nn.relu(jnp.dot(du[:, :D], w0_ref[:D], preferred_element_type=jnp.float32)
                    + jnp.dot(di[:, :D], w0_ref[D:], preferred_element_type=jnp.float32)
                    + b0_ref[...])
    x = jax.nn.relu(jnp.dot(x, w1_ref[...], preferred_element_type=jnp.float32)
                    + b1_ref[...])
    x = jax.nn.relu(jnp.dot(x, w2_ref[...], preferred_element_type=jnp.float32)
                    + b2_ref[...])
    deep = jnp.dot(x, w3_ref[...], preferred_element_type=jnp.float32)
    out_ref[...] = deep[:, 0] + b3_ref[0, 0] + du[:, D] + di[:, D]


def _mlp_call(du, di, w0t, b0, w1, b1, w2, b2, w3, b3):
    full = lambda shape: pl.BlockSpec(shape, lambda i: (0,) * len(shape))
    return pl.pallas_call(
        _mlp_body,
        grid=(B // BLK,),
        in_specs=[
            pl.BlockSpec((BLK, 2 * D), lambda i: (i, 0)),
            pl.BlockSpec((BLK, 2 * D), lambda i: (i, 0)),
            full((128, 128)),
            full((1, 128)),
            full((128, 64)),
            full((1, 64)),
            full((64, 32)),
            full((1, 32)),
            full((32, 1)),
            full((1, 1)),
        ],
        out_specs=pl.BlockSpec((BLK,), lambda i: (i,)),
        out_shape=jax.ShapeDtypeStruct((B,), jnp.float32),
    )(du, di, w0t, b0, w1, b1, w2, b2, w3, b3)


def kernel(user_ids, item_ids, wide_user, wide_item, deep_user, deep_item,
           W0, b0, W1, b1, W2, b2, W3, b3):
    uids = user_ids.astype(jnp.int32)
    iids = item_ids.astype(jnp.int32)
    pos = lax.iota(jnp.int32, B)
    su, pu = lax.sort_key_val(uids, pos)
    si, pi = lax.sort_key_val(iids, pos)
    du, di = _build_sc_deep()(su, pu, si, pi,
                              deep_user.T, deep_item.T,
                              wide_user.T, wide_item.T)
    return _mlp_call(
        du, di,
        W0.T, b0.reshape(1, -1),
        W1.T, b1.reshape(1, -1),
        W2.T, b2.reshape(1, -1),
        W3.T, b3.reshape(1, 1),
    )


# speedup vs baseline: 2.0633x; 1.1115x over previous
import functools

import jax
import jax.numpy as jnp
from jax import lax
from jax.experimental import pallas as pl
from jax.experimental.pallas import tpu as pltpu
from jax.experimental.pallas import tpu_sc as plsc

B = 16384
D = 64
NC = 2
NS = 16
NW = NC * NS
BPW = B // NW
CH = 128
L = 16
NBUF = 6
BLK = 512


def _pass_body(sids, perm, deep_t, wide_t, row_out,
               sid_vm, perm_vm, blk_s, sid_s, slot, wslot, row_v,
               sem_arr, base):
    pltpu.sync_copy(sids.at[pl.ds(base, BPW)], sid_vm)
    for j in range(BPW // CH):
        pltpu.sync_copy(perm.at[pl.ds(base + j * CH, CH)], perm_vm.at[j])

    def fill(k, carry):
        v16 = sid_vm[pl.ds(k * L, L)]
        for l in range(L):
            sid_s[k * L + l] = v16[l]
        return carry

    lax.fori_loop(0, BPW // L, fill, 0)

    def scan(b, carry):
        prev, nu = carry
        blk = lax.shift_right_logical(sid_s[b], 7)
        is_new = jnp.logical_or(b == 0, blk != prev)
        nu_new = jnp.where(is_new, nu + 1, nu)

        @pl.when(is_new)
        def _record():
            blk_s[nu] = blk

        return blk, nu_new

    _, nu = lax.fori_loop(0, BPW, scan, (jnp.int32(-1), jnp.int32(0)))

    def fire(k):
        off = pl.multiple_of(blk_s[k] * CH, CH)
        s = jnp.remainder(k, NBUF)
        pltpu.async_copy(deep_t.at[:, pl.ds(off, CH)], slot.at[s], sem_arr.at[s])
        pltpu.async_copy(wide_t.at[:, pl.ds(off, CH)], wslot.at[s], sem_arr.at[s])

    for j in range(NBUF):
        @pl.when(j < nu)
        def _prime():
            fire(jnp.int32(j))

    def body(b, carry):
        k, prev = carry
        blk = lax.shift_right_logical(sid_s[b], 7)
        is_new = jnp.logical_or(b == 0, blk != prev)
        k = jnp.where(is_new, k + 1, k)
        s = jnp.remainder(k, NBUF)

        @pl.when(is_new)
        def _advance():
            @pl.when(jnp.logical_and(k > 0, k + NBUF - 1 < nu))
            def _refill():
                fire(k + NBUF - 1)

            pltpu.make_async_copy(deep_t.at[:, pl.ds(0, CH)], slot.at[s], sem_arr.at[s]).wait()
            pltpu.make_async_copy(wide_t.at[:, pl.ds(0, CH)], wslot.at[s], sem_arr.at[s]).wait()

        c = jnp.bitwise_and(sid_s[b], CH - 1)
        cols = jnp.full((L,), c, jnp.int32)
        for kk in range(D // L):
            rows = lax.iota(jnp.int32, L) + (kk * L)
            row_v[b, pl.ds(kk * L, L)] = plsc.load_gather(slot.at[s], [rows, cols])
        wv = plsc.load_gather(wslot.at[s], [jnp.zeros((L,), jnp.int32), cols])
        row_v[b, pl.ds(D, L)] = wv
        return k, blk

    lax.fori_loop(0, BPW, body, (jnp.int32(-1), jnp.int32(-1)))

    copies = []
    for j in range(BPW // CH):
        copies.append(pltpu.async_copy(
            row_v.at[pl.ds(j * CH, CH)], row_out.at[perm_vm.at[j]], sem_arr.at[j]))
    for c in copies:
        c.wait()


def _sc_deep_body(su, pu, si, pi, deep_u, deep_i, wide_u, wide_i,
                  du_out, di_out,
                  sid_vm, perm_vm, blk_s, sid_s, slot, wslot, row_v, sem_arr):
    wid = lax.axis_index("s") * NC + lax.axis_index("c")
    base = wid * BPW
    _pass_body(su, pu, deep_u, wide_u, du_out,
               sid_vm, perm_vm, blk_s, sid_s, slot, wslot, row_v, sem_arr, base)
    _pass_body(si, pi, deep_i, wide_i, di_out,
               sid_vm, perm_vm, blk_s, sid_s, slot, wslot, row_v, sem_arr, base)


@functools.lru_cache(maxsize=1)
def _build_sc_deep():
    return functools.partial(
        pl.kernel,
        out_type=(
            jax.ShapeDtypeStruct((B, 2 * D), jnp.float32),
            jax.ShapeDtypeStruct((B, 2 * D), jnp.float32),
        ),
        mesh=plsc.VectorSubcoreMesh(
            core_axis_name="c", subcore_axis_name="s", num_cores=NC, num_subcores=NS
        ),
        scratch_types=(
            pltpu.VMEM((BPW,), jnp.int32),
            pltpu.VMEM((BPW // CH, CH), jnp.int32),
            pltpu.SMEM((BPW,), jnp.int32),
            pltpu.SMEM((BPW,), jnp.int32),
            pltpu.VMEM((NBUF, D, CH), jnp.float32),
            pltpu.VMEM((NBUF, 1, CH), jnp.float32),
            pltpu.VMEM((BPW, 2 * D), jnp.float32),
            pltpu.SemaphoreType.DMA((NBUF,)),
        ),
        compiler_params=pltpu.CompilerParams(
            use_tc_tiling_on_sc=True, needs_layout_passes=False),
    )(_sc_deep_body)


def _mlp_body(du_ref, di_ref, w0_ref, b0_ref, w1_ref, b1_ref,
              w2_ref, b2_ref, w3_ref, b3_ref, out_ref):
    du = du_ref[...]
    di = di_ref[...]
    x = jax.nn.relu(jnp.dot(du[:, :D], w0_ref[:D], preferred_element_type=jnp.float32)
                    + jnp.dot(di[:, :D], w0_ref[D:], preferred_element_type=jnp.float32)
                    + b0_ref[...])
    x = jax.nn.relu(jnp.dot(x, w1_ref[...], preferred_element_type=jnp.float32)
                    + b1_ref[...])
    x = jax.nn.relu(jnp.dot(x, w2_ref[...], preferred_element_type=jnp.float32)
                    + b2_ref[...])
    deep = jnp.dot(x, w3_ref[...], preferred_element_type=jnp.float32)
    out_ref[...] = deep[:, 0] + b3_ref[0, 0] + du[:, D] + di[:, D]


def _mlp_call(du, di, w0t, b0, w1, b1, w2, b2, w3, b3):
    full = lambda shape: pl.BlockSpec(shape, lambda i: (0,) * len(shape))
    return pl.pallas_call(
        _mlp_body,
        grid=(B // BLK,),
        in_specs=[
            pl.BlockSpec((BLK, 2 * D), lambda i: (i, 0)),
            pl.BlockSpec((BLK, 2 * D), lambda i: (i, 0)),
            full((128, 128)),
            full((1, 128)),
            full((128, 64)),
            full((1, 64)),
            full((64, 32)),
            full((1, 32)),
            full((32, 1)),
            full((1, 1)),
        ],
        out_specs=pl.BlockSpec((BLK,), lambda i: (i,)),
        out_shape=jax.ShapeDtypeStruct((B,), jnp.float32),
    )(du, di, w0t, b0, w1, b1, w2, b2, w3, b3)


def kernel(user_ids, item_ids, wide_user, wide_item, deep_user, deep_item,
           W0, b0, W1, b1, W2, b2, W3, b3):
    uids = user_ids.astype(jnp.int32)
    iids = item_ids.astype(jnp.int32)
    pos = lax.iota(jnp.int32, B)
    su, pu = lax.sort_key_val(uids, pos)
    si, pi = lax.sort_key_val(iids, pos)
    du, di = _build_sc_deep()(su, pu, si, pi,
                              deep_user.T, deep_item.T,
                              wide_user.T, wide_item.T)
    return _mlp_call(
        du, di,
        W0.T, b0.reshape(1, -1),
        W1.T, b1.reshape(1, -1),
        W2.T, b2.reshape(1, -1),
        W3.T, b3.reshape(1, 1),
    )


# speedup vs baseline: 2.0897x; 1.0128x over previous
import functools

import jax
import jax.numpy as jnp
from jax import lax
from jax.experimental import pallas as pl
from jax.experimental.pallas import tpu as pltpu
from jax.experimental.pallas import tpu_sc as plsc

B = 16384
D = 64
NC = 2
NS = 16
NW = NC * NS
BPW = B // NW
CH = 128
L = 16
NBUF = 7
BLK = 512


def _pass_body(sids, perm, deep_t, wide_t, row_out,
               sid_vm, perm_vm, blk_s, sid_s, slot, wslot, row_v,
               sem_arr, base):
    pltpu.sync_copy(sids.at[pl.ds(base, BPW)], sid_vm)
    for j in range(BPW // CH):
        pltpu.sync_copy(perm.at[pl.ds(base + j * CH, CH)], perm_vm.at[j])

    def fill(k, carry):
        v16 = sid_vm[pl.ds(k * L, L)]
        for l in range(L):
            sid_s[k * L + l] = v16[l]
        return carry

    lax.fori_loop(0, BPW // L, fill, 0)

    def scan(b, carry):
        prev, nu = carry
        blk = lax.shift_right_logical(sid_s[b], 7)
        is_new = jnp.logical_or(b == 0, blk != prev)
        nu_new = jnp.where(is_new, nu + 1, nu)

        @pl.when(is_new)
        def _record():
            blk_s[nu] = blk

        return blk, nu_new

    _, nu = lax.fori_loop(0, BPW, scan, (jnp.int32(-1), jnp.int32(0)))

    def fire(k):
        off = pl.multiple_of(blk_s[k] * CH, CH)
        s = jnp.remainder(k, NBUF)
        pltpu.async_copy(deep_t.at[:, pl.ds(off, CH)], slot.at[s], sem_arr.at[s])
        pltpu.async_copy(wide_t.at[:, pl.ds(off, CH)], wslot.at[s], sem_arr.at[s])

    for j in range(NBUF):
        @pl.when(j < nu)
        def _prime():
            fire(jnp.int32(j))

    def body(b, carry):
        k, prev = carry
        blk = lax.shift_right_logical(sid_s[b], 7)
        is_new = jnp.logical_or(b == 0, blk != prev)
        k = jnp.where(is_new, k + 1, k)
        s = jnp.remainder(k, NBUF)

        @pl.when(is_new)
        def _advance():
            @pl.when(jnp.logical_and(k > 0, k + NBUF - 1 < nu))
            def _refill():
                fire(k + NBUF - 1)

            pltpu.make_async_copy(deep_t.at[:, pl.ds(0, CH)], slot.at[s], sem_arr.at[s]).wait()
            pltpu.make_async_copy(wide_t.at[:, pl.ds(0, CH)], wslot.at[s], sem_arr.at[s]).wait()

        c = jnp.bitwise_and(sid_s[b], CH - 1)
        cols = jnp.full((L,), c, jnp.int32)
        for kk in range(D // L):
            rows = lax.iota(jnp.int32, L) + (kk * L)
            row_v[b, pl.ds(kk * L, L)] = plsc.load_gather(slot.at[s], [rows, cols])
        wv = plsc.load_gather(wslot.at[s], [jnp.zeros((L,), jnp.int32), cols])
        row_v[b, pl.ds(D, L)] = wv
        return k, blk

    lax.fori_loop(0, BPW, body, (jnp.int32(-1), jnp.int32(-1)))

    copies = []
    for j in range(BPW // CH):
        copies.append(pltpu.async_copy(
            row_v.at[pl.ds(j * CH, CH)], row_out.at[perm_vm.at[j]], sem_arr.at[j]))
    for c in copies:
        c.wait()


def _sc_deep_body(su, pu, si, pi, deep_u, deep_i, wide_u, wide_i,
                  du_out, di_out,
                  sid_vm, perm_vm, blk_s, sid_s, slot, wslot, row_v, sem_arr):
    wid = lax.axis_index("s") * NC + lax.axis_index("c")
    base = wid * BPW
    _pass_body(su, pu, deep_u, wide_u, du_out,
               sid_vm, perm_vm, blk_s, sid_s, slot, wslot, row_v, sem_arr, base)
    _pass_body(si, pi, deep_i, wide_i, di_out,
               sid_vm, perm_vm, blk_s, sid_s, slot, wslot, row_v, sem_arr, base)


@functools.lru_cache(maxsize=1)
def _build_sc_deep():
    return functools.partial(
        pl.kernel,
        out_type=(
            jax.ShapeDtypeStruct((B, 2 * D), jnp.float32),
            jax.ShapeDtypeStruct((B, 2 * D), jnp.float32),
        ),
        mesh=plsc.VectorSubcoreMesh(
            core_axis_name="c", subcore_axis_name="s", num_cores=NC, num_subcores=NS
        ),
        scratch_types=(
            pltpu.VMEM((BPW,), jnp.int32),
            pltpu.VMEM((BPW // CH, CH), jnp.int32),
            pltpu.SMEM((BPW,), jnp.int32),
            pltpu.SMEM((BPW,), jnp.int32),
            pltpu.VMEM((NBUF, D, CH), jnp.float32),
            pltpu.VMEM((NBUF, 1, CH), jnp.float32),
            pltpu.VMEM((BPW, 2 * D), jnp.float32),
            pltpu.SemaphoreType.DMA((NBUF,)),
        ),
        compiler_params=pltpu.CompilerParams(
            use_tc_tiling_on_sc=True, needs_layout_passes=False),
    )(_sc_deep_body)


def _mlp_body(du_ref, di_ref, w0_ref, b0_ref, w1_ref, b1_ref,
              w2_ref, b2_ref, w3_ref, b3_ref, out_ref):
    du = du_ref[...]
    di = di_ref[...]
    x = jax.nn.relu(jnp.dot(du[:, :D], w0_ref[:D], preferred_element_type=jnp.float32)
                    + jnp.dot(di[:, :D], w0_ref[D:], preferred_element_type=jnp.float32)
                    + b0_ref[...])
    x = jax.nn.relu(jnp.dot(x, w1_ref[...], preferred_element_type=jnp.float32)
                    + b1_ref[...])
    x = jax.nn.relu(jnp.dot(x, w2_ref[...], preferred_element_type=jnp.float32)
                    + b2_ref[...])
    deep = jnp.dot(x, w3_ref[...], preferred_element_type=jnp.float32)
    out_ref[...] = deep[:, 0] + b3_ref[0, 0] + du[:, D] + di[:, D]


def _mlp_call(du, di, w0t, b0, w1, b1, w2, b2, w3, b3):
    full = lambda shape: pl.BlockSpec(shape, lambda i: (0,) * len(shape))
    return pl.pallas_call(
        _mlp_body,
        grid=(B // BLK,),
        in_specs=[
            pl.BlockSpec((BLK, 2 * D), lambda i: (i, 0)),
            pl.BlockSpec((BLK, 2 * D), lambda i: (i, 0)),
            full((128, 128)),
            full((1, 128)),
            full((128, 64)),
            full((1, 64)),
            full((64, 32)),
            full((1, 32)),
            full((32, 1)),
            full((1, 1)),
        ],
        out_specs=pl.BlockSpec((BLK,), lambda i: (i,)),
        out_shape=jax.ShapeDtypeStruct((B,), jnp.float32),
    )(du, di, w0t, b0, w1, b1, w2, b2, w3, b3)


def kernel(user_ids, item_ids, wide_user, wide_item, deep_user, deep_item,
           W0, b0, W1, b1, W2, b2, W3, b3):
    uids = user_ids.astype(jnp.int32)
    iids = item_ids.astype(jnp.int32)
    pos = lax.iota(jnp.int32, B)
    su, pu = lax.sort_key_val(uids, pos)
    si, pi = lax.sort_key_val(iids, pos)
    du, di = _build_sc_deep()(su, pu, si, pi,
                              deep_user.T, deep_item.T,
                              wide_user.T, wide_item.T)
    return _mlp_call(
        du, di,
        W0.T, b0.reshape(1, -1),
        W1.T, b1.reshape(1, -1),
        W2.T, b2.reshape(1, -1),
        W3.T, b3.reshape(1, 1),
    )
